# Initial kernel scaffold; baseline (speedup 1.0000x reference)
import jax
import jax.numpy as jnp
from jax.experimental import pallas as pl


def kernel(input, weight, bias):
    raise NotImplementedError("write your pallas kernel here")


# speedup vs baseline: 31.5684x; 31.5684x over previous
import jax
import jax.numpy as jnp
from jax.experimental import pallas as pl
from jax.experimental.pallas import tpu as pltpu

_EPS = 1e-10
_H, _W = 1024, 1024
_BR = 128
_NB = _H // _BR


def _rb(a):
    return a.astype(jnp.bfloat16).astype(jnp.float32)


def _mt_kernel(params_ref, x_ref, out_ref, carry_ref, prevq_ref):
    i = pl.program_id(0)

    @pl.when(i == 0)
    def _init():
        carry_ref[...] = jnp.zeros_like(carry_ref)
        prevq_ref[...] = jnp.zeros_like(prevq_ref)

    x = x_ref[...]
    q = jnp.round(x)
    absv = jnp.abs(q)
    la = jnp.log(absv + _EPS)
    t = jnp.exp(0.1 * la)
    lb = jnp.log(1.0 + absv + _EPS)
    ls = jnp.log(0.5 + 0.5 * absv + _EPS)

    r = jax.lax.broadcasted_iota(jnp.int32, (_BR, _W), 0).astype(jnp.float32) + (i * _BR).astype(jnp.float32)
    c = jax.lax.broadcasted_iota(jnp.int32, (_BR, _W), 1).astype(jnp.float32)
    s = r + c

    logit = (params_ref[0]
             + _rb(c) * params_ref[1]
             + _rb(r) * params_ref[2]
             + _rb(jnp.float32(_H) - r) * params_ref[3]
             + _rb(lb) * params_ref[4]
             + _rb(ls) * params_ref[5])
    tp = t
    for k in range(1, 8):
        if k > 1:
            tp = tp * t
        logit += params_ref[5 + k] * _rb(jnp.log(tp + (0.01 * k) * s + _EPS))

    cx1 = c + 1.0
    rx1 = r + 1.0
    n = jax.lax.rsqrt(cx1 * cx1 + rx1 * rx1)
    ca = cx1 * n
    sa = rx1 * n
    cv = jnp.cos(0.1 * q)
    sv = jnp.sin(0.1 * q)
    logit += _rb(ca * cv - sa * sv) * params_ref[13]
    logit += _rb(sa * cv + ca * sv) * params_ref[14]

    score = jax.nn.sigmoid(logit)

    prev = prevq_ref[...]
    qshift = jnp.concatenate([prev, q[:-1, :]], axis=0)
    m = (q - qshift) * score

    ri = jax.lax.broadcasted_iota(jnp.int32, (_BR, _BR), 0)
    ci = jax.lax.broadcasted_iota(jnp.int32, (_BR, _BR), 1)
    ltri = (ci <= ri).astype(jnp.float32)
    cum = jax.lax.dot(ltri, m, precision=jax.lax.Precision.HIGHEST,
                      preferred_element_type=jnp.float32)

    out = cum + carry_ref[...]
    out_ref[...] = out
    carry_ref[...] = out[_BR - 1:_BR, :]
    prevq_ref[...] = q[_BR - 1:_BR, :]


def kernel(input, weight, bias):
    wb = weight[:, 0].astype(jnp.bfloat16).astype(jnp.float32)
    k0 = wb[14] * jnp.float32(0.70703125) + bias[0]
    params = jnp.concatenate([
        jnp.stack([k0,
                   wb[0] + wb[2],
                   wb[1] + wb[3],
                   wb[4],
                   wb[5],
                   wb[6]]),
        wb[7:14],
        wb[15:17],
    ]).astype(jnp.float32)

    return pl.pallas_call(
        _mt_kernel,
        grid=(_NB,),
        in_specs=[
            pl.BlockSpec(memory_space=pltpu.SMEM),
            pl.BlockSpec((_BR, _W), lambda i: (i, 0)),
        ],
        out_specs=pl.BlockSpec((_BR, _W), lambda i: (i, 0)),
        out_shape=jax.ShapeDtypeStruct((_H, _W), jnp.float32),
        scratch_shapes=[
            pltpu.VMEM((1, _W), jnp.float32),
            pltpu.VMEM((1, _W), jnp.float32),
        ],
    )(params, input)
